# Initial kernel scaffold; baseline (speedup 1.0000x reference)
import jax
import jax.numpy as jnp
from jax.experimental import pallas as pl


def kernel(nodes, batch_idx, W_gate, b_gate, W_attn, b_attn):
    raise NotImplementedError("write your pallas kernel here")


# speedup vs baseline: 1.5386x; 1.5386x over previous
import functools

import jax
import jax.numpy as jnp
from jax import lax
from jax.experimental import pallas as pl
from jax.experimental.pallas import tpu as pltpu
from jax.experimental.pallas import tpu_sc as plsc

N = 50000
D = 512
S = 512

_FMIN = jnp.finfo(jnp.float32).min

_RA = 400
_NB = N // _RA

_NC = 2
_NS = 16
_RSH = 8
_CG = 4
_CGW = D // _CG
_CH = 64
_PER_SH = 6272
_NPAD = _RSH * _PER_SH
_NCH = _PER_SH // _CH

_RA2 = 448
_NB2 = _NPAD // _RA2


def _gate_body(nodes_ref, wg_ref, bg_ref, idx_ref, x_ref, segmax_ref):
    i = pl.program_id(0)
    xb = jnp.dot(nodes_ref[...], wg_ref[...],
                 preferred_element_type=jnp.float32) + bg_ref[0, 0]
    x_ref[...] = xb
    cols = lax.broadcasted_iota(jnp.int32, (_RA, S), 1)
    oh = idx_ref[...] == cols
    bmax = jnp.max(jnp.where(oh, xb, _FMIN), axis=0, keepdims=True)

    @pl.when(i == 0)
    def _init():
        segmax_ref[...] = jnp.full((1, S), _FMIN, jnp.float32)

    segmax_ref[...] = jnp.maximum(segmax_ref[...], bmax)


def _gate_call(nodes, w_gate, b_gate, idx2d):
    return pl.pallas_call(
        _gate_body,
        grid=(_NB,),
        in_specs=[
            pl.BlockSpec((_RA, D), lambda i: (i, 0)),
            pl.BlockSpec((D, 1), lambda i: (0, 0)),
            pl.BlockSpec((1, 1), lambda i: (0, 0)),
            pl.BlockSpec((_RA, 1), lambda i: (i, 0)),
        ],
        out_specs=[
            pl.BlockSpec((_RA, 1), lambda i: (i, 0)),
            pl.BlockSpec((1, S), lambda i: (0, 0)),
        ],
        out_shape=[
            jax.ShapeDtypeStruct((N, 1), jnp.float32),
            jax.ShapeDtypeStruct((1, S), jnp.float32),
        ],
    )(nodes, w_gate, b_gate, idx2d)


def _sumexp_body(x_ref, idx_ref, segmax_ref, sumexp_ref):
    j = pl.program_id(0)
    xb = x_ref[...]
    cols = lax.broadcasted_iota(jnp.int32, (_RA, S), 1)
    oh = idx_ref[...] == cols
    m = jnp.max(jnp.where(oh, segmax_ref[...], _FMIN), axis=1,
                keepdims=True)
    e = jnp.exp(xb - m)

    @pl.when(j == 0)
    def _init():
        sumexp_ref[...] = jnp.zeros((1, S), jnp.float32)

    sumexp_ref[...] += jnp.sum(jnp.where(oh, e, 0.0), axis=0, keepdims=True)


def _sumexp_call(x, idx2d, segmax):
    return pl.pallas_call(
        _sumexp_body,
        grid=(_NB,),
        in_specs=[
            pl.BlockSpec((_RA, 1), lambda j: (j, 0)),
            pl.BlockSpec((_RA, 1), lambda j: (j, 0)),
            pl.BlockSpec((1, S), lambda j: (0, 0)),
        ],
        out_specs=pl.BlockSpec((1, S), lambda j: (0, 0)),
        out_shape=jax.ShapeDtypeStruct((1, S), jnp.float32),
    )(x, idx2d, segmax)


def _alpha_body(x_ref, idx_ref, segmax_ref, sumexp_ref, alpha_ref, seg_ref):
    xb = x_ref[...]
    idx = idx_ref[...]
    cols = lax.broadcasted_iota(jnp.int32, (_RA2, S), 1)
    oh = idx == cols
    m = jnp.max(jnp.where(oh, segmax_ref[...], _FMIN), axis=1,
                keepdims=True)
    e = jnp.exp(xb - m)
    den = jnp.max(jnp.where(oh, sumexp_ref[...] + 1e-16, 0.0), axis=1,
                  keepdims=True)
    alpha_ref[...] = jnp.broadcast_to(e / den, (_RA2, 16))
    seg_ref[...] = jnp.broadcast_to(idx, (_RA2, 16))


def _alpha_call(x_pad, idx_pad2d, segmax, sumexp):
    return pl.pallas_call(
        _alpha_body,
        grid=(_NB2,),
        in_specs=[
            pl.BlockSpec((_RA2, 1), lambda j: (j, 0)),
            pl.BlockSpec((_RA2, 1), lambda j: (j, 0)),
            pl.BlockSpec((1, S), lambda j: (0, 0)),
            pl.BlockSpec((1, S), lambda j: (0, 0)),
        ],
        out_specs=[
            pl.BlockSpec((_RA2, 16), lambda j: (j, 0)),
            pl.BlockSpec((_RA2, 16), lambda j: (j, 0)),
        ],
        out_shape=[
            jax.ShapeDtypeStruct((_NPAD, 16), jnp.float32),
            jax.ShapeDtypeStruct((_NPAD, 16), jnp.int32),
        ],
    )(x_pad, idx_pad2d, segmax, sumexp)


def _pool_body(nodes_hbm, alpha_hbm, seg_hbm, out_hbm,
               rows_v, alpha_v, seg_v, rowidx_v, acc, sem):
    c = lax.axis_index("c")
    s = lax.axis_index("s")
    w = c * _NS + s
    rsh = w % _RSH
    cg = w // _RSH
    col0 = cg * _CGW

    zero16 = jnp.zeros((16,), jnp.float32)

    def _zero_row(r, carry):
        for cc in range(_CGW // 16):
            acc[r, pl.ds(cc * 16, 16)] = zero16
        return carry

    lax.fori_loop(0, S, _zero_row, 0)

    iota16 = lax.broadcasted_iota(jnp.int32, (16,), 0)

    def _chunk(jc, carry):
        base = rsh * _PER_SH + jc * _CH
        for t in range(_CH // 16):
            rowidx_v[pl.ds(t * 16, 16)] = jnp.minimum(
                base + t * 16 + iota16, N - 1)
        pltpu.sync_copy(alpha_hbm.at[pl.ds(base, _CH), :], alpha_v)
        pltpu.sync_copy(seg_hbm.at[pl.ds(base, _CH), :], seg_v)
        pltpu.async_copy(nodes_hbm.at[rowidx_v, pl.ds(col0, _CGW)],
                         rows_v, sem).wait()

        def _row(r, carry2):
            a16 = alpha_v[r, pl.ds(0, 16)]
            s16 = seg_v[r, pl.ds(0, 16)]
            for cc in range(_CGW // 16):
                val = rows_v[r, pl.ds(cc * 16, 16)] * a16
                plsc.addupdate_scatter(acc, [s16, cc * 16 + iota16], val)
            return carry2

        lax.fori_loop(0, _CH, _row, 0)
        return carry

    lax.fori_loop(0, _NCH, _chunk, 0)
    pltpu.sync_copy(acc, out_hbm.at[rsh, :, pl.ds(col0, _CGW)])


def _pool_call(nodes, alpha16, seg16):
    mesh = plsc.VectorSubcoreMesh(core_axis_name="c", subcore_axis_name="s",
                                  num_cores=_NC, num_subcores=_NS)
    fn = pl.kernel(
        _pool_body,
        out_type=jax.ShapeDtypeStruct((_RSH, S, D), jnp.float32),
        mesh=mesh,
        compiler_params=pltpu.CompilerParams(needs_layout_passes=False),
        scratch_types=[
            pltpu.VMEM((_CH, _CGW), jnp.float32),
            pltpu.VMEM((_CH, 16), jnp.float32),
            pltpu.VMEM((_CH, 16), jnp.int32),
            pltpu.VMEM((_CH,), jnp.int32),
            pltpu.VMEM((S, _CGW), jnp.float32),
            pltpu.SemaphoreType.DMA,
        ],
    )
    return fn(nodes, alpha16, seg16)


def _final_body(pooled_ref, c_ref, wa_ref, ba_ref, out_ref):
    p = pooled_ref[0]
    for k in range(1, _RSH):
        p = p + pooled_ref[k]
    out_ref[...] = (jnp.dot(p, wa_ref[...], preferred_element_type=jnp.float32)
                    + c_ref[...] * ba_ref[...])


def _final_call(pooled, c_col, w_attn, b_attn_row):
    return pl.pallas_call(
        _final_body,
        out_shape=jax.ShapeDtypeStruct((S, D), jnp.float32),
    )(pooled, c_col, w_attn, b_attn_row)


def kernel(nodes, batch_idx, W_gate, b_gate, W_attn, b_attn):
    idx32 = batch_idx.astype(jnp.int32)
    idx2d = idx32.reshape(N, 1)
    x, segmax = _gate_call(nodes, W_gate, b_gate.reshape(1, 1), idx2d)
    sumexp = _sumexp_call(x, idx2d, segmax)
    x_pad = jnp.concatenate(
        [x, jnp.full((_NPAD - N, 1), -jnp.inf, jnp.float32)])
    idx_pad2d = jnp.concatenate(
        [idx2d, jnp.zeros((_NPAD - N, 1), jnp.int32)])
    alpha16, seg16 = _alpha_call(x_pad, idx_pad2d, segmax, sumexp)
    pooled = _pool_call(nodes, alpha16, seg16)
    c_col = (sumexp / (sumexp + 1e-16)).reshape(S, 1)
    return _final_call(pooled, c_col, W_attn, b_attn.reshape(1, D))


# speedup vs baseline: 2.6707x; 1.7358x over previous
import functools

import jax
import jax.numpy as jnp
from jax import lax
from jax.experimental import pallas as pl
from jax.experimental.pallas import tpu as pltpu
from jax.experimental.pallas import tpu_sc as plsc

N = 50000
D = 512
S = 512

_FMIN = jnp.finfo(jnp.float32).min

_RA = 400
_NB = N // _RA

_NC = 2
_NS = 16
_RSH = 8
_CG = 4
_CGW = D // _CG
_CH = 64
_PER_SH = 6272
_NPAD = _RSH * _PER_SH
_NCH = _PER_SH // _CH

_RA2 = 448
_NB2 = _NPAD // _RA2


def _gate_body(nodes_ref, wg_ref, bg_ref, idx_ref, x_ref, segmax_ref):
    i = pl.program_id(0)
    xb = jnp.dot(nodes_ref[...], wg_ref[...],
                 preferred_element_type=jnp.float32) + bg_ref[0, 0]
    x_ref[...] = xb
    cols = lax.broadcasted_iota(jnp.int32, (_RA, S), 1)
    oh = idx_ref[...] == cols
    bmax = jnp.max(jnp.where(oh, xb, _FMIN), axis=0, keepdims=True)

    @pl.when(i == 0)
    def _init():
        segmax_ref[...] = jnp.full((1, S), _FMIN, jnp.float32)

    segmax_ref[...] = jnp.maximum(segmax_ref[...], bmax)


def _gate_call(nodes, w_gate, b_gate, idx2d):
    return pl.pallas_call(
        _gate_body,
        grid=(_NB,),
        in_specs=[
            pl.BlockSpec((_RA, D), lambda i: (i, 0)),
            pl.BlockSpec((D, 1), lambda i: (0, 0)),
            pl.BlockSpec((1, 1), lambda i: (0, 0)),
            pl.BlockSpec((_RA, 1), lambda i: (i, 0)),
        ],
        out_specs=[
            pl.BlockSpec((_RA, 1), lambda i: (i, 0)),
            pl.BlockSpec((1, S), lambda i: (0, 0)),
        ],
        out_shape=[
            jax.ShapeDtypeStruct((N, 1), jnp.float32),
            jax.ShapeDtypeStruct((1, S), jnp.float32),
        ],
    )(nodes, w_gate, b_gate, idx2d)


def _sumexp_body(x_ref, idx_ref, segmax_ref, sumexp_ref):
    j = pl.program_id(0)
    xb = x_ref[...]
    cols = lax.broadcasted_iota(jnp.int32, (_RA, S), 1)
    oh = idx_ref[...] == cols
    m = jnp.max(jnp.where(oh, segmax_ref[...], _FMIN), axis=1,
                keepdims=True)
    e = jnp.exp(xb - m)

    @pl.when(j == 0)
    def _init():
        sumexp_ref[...] = jnp.zeros((1, S), jnp.float32)

    sumexp_ref[...] += jnp.sum(jnp.where(oh, e, 0.0), axis=0, keepdims=True)


def _sumexp_call(x, idx2d, segmax):
    return pl.pallas_call(
        _sumexp_body,
        grid=(_NB,),
        in_specs=[
            pl.BlockSpec((_RA, 1), lambda j: (j, 0)),
            pl.BlockSpec((_RA, 1), lambda j: (j, 0)),
            pl.BlockSpec((1, S), lambda j: (0, 0)),
        ],
        out_specs=pl.BlockSpec((1, S), lambda j: (0, 0)),
        out_shape=jax.ShapeDtypeStruct((1, S), jnp.float32),
    )(x, idx2d, segmax)


def _alpha_body(x_ref, idx_ref, segmax_ref, sumexp_ref, alpha_ref, seg_ref):
    xb = x_ref[...]
    idx = idx_ref[...]
    cols = lax.broadcasted_iota(jnp.int32, (_RA2, S), 1)
    oh = idx == cols
    m = jnp.max(jnp.where(oh, segmax_ref[...], _FMIN), axis=1,
                keepdims=True)
    e = jnp.exp(xb - m)
    den = jnp.max(jnp.where(oh, sumexp_ref[...] + 1e-16, 0.0), axis=1,
                  keepdims=True)
    alpha_ref[...] = jnp.broadcast_to(e / den, (_RA2, 16))
    seg_ref[...] = jnp.broadcast_to(idx, (_RA2, 16))


def _alpha_call(x_pad, idx_pad2d, segmax, sumexp):
    return pl.pallas_call(
        _alpha_body,
        grid=(_NB2,),
        in_specs=[
            pl.BlockSpec((_RA2, 1), lambda j: (j, 0)),
            pl.BlockSpec((_RA2, 1), lambda j: (j, 0)),
            pl.BlockSpec((1, S), lambda j: (0, 0)),
            pl.BlockSpec((1, S), lambda j: (0, 0)),
        ],
        out_specs=[
            pl.BlockSpec((_RA2, 16), lambda j: (j, 0)),
            pl.BlockSpec((_RA2, 16), lambda j: (j, 0)),
        ],
        out_shape=[
            jax.ShapeDtypeStruct((_NPAD, 16), jnp.float32),
            jax.ShapeDtypeStruct((_NPAD, 16), jnp.int32),
        ],
    )(x_pad, idx_pad2d, segmax, sumexp)


def _pool_body(nodes_hbm, alpha_hbm, seg_hbm, out_hbm,
               rows0, rows1, alpha0, alpha1, seg0, seg1, ridx0, ridx1,
               acc, sem0, sem1):
    c = lax.axis_index("c")
    s = lax.axis_index("s")
    w = c * _NS + s
    rsh = w % _RSH
    cg = w // _RSH
    col0 = cg * _CGW
    shard0 = rsh * _PER_SH

    zero16 = jnp.zeros((16,), jnp.float32)

    def _zero_row(r, carry):
        for cc in range(_CGW // 16):
            acc[r, pl.ds(cc * 16, 16)] = zero16
        return carry

    lax.fori_loop(0, S, _zero_row, 0)

    iota16 = lax.broadcasted_iota(jnp.int32, (16,), 0)
    bufs = ((rows0, alpha0, seg0, ridx0, sem0),
            (rows1, alpha1, seg1, ridx1, sem1))

    def _issue(b, jc):
        rows_b, alpha_b, seg_b, ridx_b, sem_b = bufs[b]
        base = shard0 + jc * _CH
        for t in range(_CH // 16):
            ridx_b[pl.ds(t * 16, 16)] = jnp.minimum(
                base + t * 16 + iota16, N - 1)
        pltpu.async_copy(alpha_hbm.at[pl.ds(base, _CH), :], alpha_b, sem_b)
        pltpu.async_copy(seg_hbm.at[pl.ds(base, _CH), :], seg_b, sem_b)
        pltpu.async_copy(nodes_hbm.at[ridx_b, pl.ds(col0, _CGW)],
                         rows_b, sem_b)

    def _wait(b):
        rows_b, alpha_b, seg_b, ridx_b, sem_b = bufs[b]
        pltpu.make_async_copy(alpha_hbm.at[pl.ds(0, _CH), :], alpha_b,
                              sem_b).wait()
        pltpu.make_async_copy(seg_hbm.at[pl.ds(0, _CH), :], seg_b,
                              sem_b).wait()
        pltpu.make_async_copy(nodes_hbm.at[pl.ds(0, _CH), pl.ds(0, _CGW)],
                              rows_b, sem_b).wait()

    def _process(b):
        rows_b, alpha_b, seg_b, ridx_b, sem_b = bufs[b]

        @plsc.parallel_loop(0, _CH, unroll=4)
        def _row(r):
            a16 = alpha_b[r, pl.ds(0, 16)]
            s16 = seg_b[r, pl.ds(0, 16)]
            for cc in range(_CGW // 16):
                val = rows_b[r, pl.ds(cc * 16, 16)] * a16
                plsc.addupdate_scatter(acc, [s16, cc * 16 + iota16], val)

    _issue(0, 0)

    def _pair(k, carry):
        j0 = 2 * k
        _issue(1, j0 + 1)
        _wait(0)
        _process(0)

        @pl.when(j0 + 2 < _NCH)
        def _prefetch():
            _issue(0, j0 + 2)

        _wait(1)
        _process(1)
        return carry

    lax.fori_loop(0, _NCH // 2, _pair, 0)
    pltpu.sync_copy(acc, out_hbm.at[rsh, :, pl.ds(col0, _CGW)])


def _pool_call(nodes, alpha16, seg16):
    mesh = plsc.VectorSubcoreMesh(core_axis_name="c", subcore_axis_name="s",
                                  num_cores=_NC, num_subcores=_NS)
    fn = pl.kernel(
        _pool_body,
        out_type=jax.ShapeDtypeStruct((_RSH, S, D), jnp.float32),
        mesh=mesh,
        compiler_params=pltpu.CompilerParams(needs_layout_passes=False),
        scratch_types=[
            pltpu.VMEM((_CH, _CGW), jnp.float32),
            pltpu.VMEM((_CH, _CGW), jnp.float32),
            pltpu.VMEM((_CH, 16), jnp.float32),
            pltpu.VMEM((_CH, 16), jnp.float32),
            pltpu.VMEM((_CH, 16), jnp.int32),
            pltpu.VMEM((_CH, 16), jnp.int32),
            pltpu.VMEM((_CH,), jnp.int32),
            pltpu.VMEM((_CH,), jnp.int32),
            pltpu.VMEM((S, _CGW), jnp.float32),
            pltpu.SemaphoreType.DMA,
            pltpu.SemaphoreType.DMA,
        ],
    )
    return fn(nodes, alpha16, seg16)


def _final_body(pooled_ref, c_ref, wa_ref, ba_ref, out_ref):
    p = pooled_ref[0]
    for k in range(1, _RSH):
        p = p + pooled_ref[k]
    out_ref[...] = (jnp.dot(p, wa_ref[...], preferred_element_type=jnp.float32)
                    + c_ref[...] * ba_ref[...])


def _final_call(pooled, c_col, w_attn, b_attn_row):
    return pl.pallas_call(
        _final_body,
        out_shape=jax.ShapeDtypeStruct((S, D), jnp.float32),
    )(pooled, c_col, w_attn, b_attn_row)


def kernel(nodes, batch_idx, W_gate, b_gate, W_attn, b_attn):
    idx32 = batch_idx.astype(jnp.int32)
    idx2d = idx32.reshape(N, 1)
    x, segmax = _gate_call(nodes, W_gate, b_gate.reshape(1, 1), idx2d)
    sumexp = _sumexp_call(x, idx2d, segmax)
    x_pad = jnp.concatenate(
        [x, jnp.full((_NPAD - N, 1), -jnp.inf, jnp.float32)])
    idx_pad2d = jnp.concatenate(
        [idx2d, jnp.zeros((_NPAD - N, 1), jnp.int32)])
    alpha16, seg16 = _alpha_call(x_pad, idx_pad2d, segmax, sumexp)
    pooled = _pool_call(nodes, alpha16, seg16)
    c_col = (sumexp / (sumexp + 1e-16)).reshape(S, 1)
    return _final_call(pooled, c_col, W_attn, b_attn.reshape(1, D))
